# baseline (device time: 16144 ns/iter reference)
import jax
import jax.numpy as jnp
from jax import lax
from jax.experimental import pallas as pl
from jax.experimental.pallas import tpu as pltpu

N_DEV = 4
_GELU_C = 0.7978845608028654

_R_TOP = 0
_R_BOT = 1
_L_BOT = 2
_L_TOP = 3
_FWD_R = 4
_FWD_L = 5


def _gelu(y):
    return 0.5 * y * (1.0 + jnp.tanh(_GELU_C * (y + 0.044715 * y * y * y)))


def kernel(x, w_mat):
    m_per, k = x.shape
    _, n_per = w_mat.shape
    half = m_per // 2

    def body(x_hbm, w_hbm, out_ref, x32_ref, w32_ref, xb_ref, wb_ref,
             xg_ref, f8s_ref, f8r_ref, cp_sems, send_sems, recv_sems):
        my = lax.axis_index("i")
        left = (my - 1) % N_DEV
        right = (my + 1) % N_DEV
        opp = (my + 2) % N_DEV

        def rows(origin, size=m_per, off=0):
            return pl.ds(origin * m_per + off, size)

        def copy(src, dst_rows, sem, dev):
            return pltpu.make_async_remote_copy(
                src_ref=src, dst_ref=xg_ref.at[dst_rows, :],
                send_sem=send_sems.at[sem], recv_sem=recv_sems.at[sem],
                device_id=(dev,), device_id_type=pl.DeviceIdType.MESH,
            )

        def gemm(x_block, origin):
            y = jnp.dot(x_block, wb_ref[:, :], preferred_element_type=jnp.float32)
            out_ref[rows(origin), :] = _gelu(y)

        cp_x = pltpu.make_async_copy(x_hbm, x32_ref, cp_sems.at[0])
        cp_w = pltpu.make_async_copy(w_hbm, w32_ref, cp_sems.at[1])
        cp_x.start()
        cp_w.start()

        barrier_sem = pltpu.get_barrier_semaphore()
        for nbr in (left, right):
            pl.semaphore_signal(
                barrier_sem, inc=1,
                device_id=(nbr,), device_id_type=pl.DeviceIdType.MESH,
            )
        pl.semaphore_wait(barrier_sem, 2)

        cp_x.wait()
        xb_ref[:, :] = x32_ref[:, :].astype(jnp.bfloat16)

        s_r_top = copy(xb_ref.at[pl.ds(0, half), :], rows(my, half), _R_TOP, right)
        s_r_bot = copy(xb_ref.at[pl.ds(half, half), :], rows(my, half, half), _R_BOT, right)
        s_l_bot = copy(xb_ref.at[pl.ds(half, half), :], rows(my, half, half), _L_BOT, left)
        s_l_top = copy(xb_ref.at[pl.ds(0, half), :], rows(my, half), _L_TOP, left)
        s_r_top.start()
        s_l_bot.start()
        s_r_bot.start()
        s_l_top.start()

        cp_w.wait()
        wb_ref[:, :] = w32_ref[:, :].astype(jnp.bfloat16)
        gemm(xb_ref[:, :], my)

        def f8copy(slot, dev):
            return pltpu.make_async_remote_copy(
                src_ref=f8s_ref.at[slot], dst_ref=f8r_ref.at[slot],
                send_sem=send_sems.at[_FWD_R + slot],
                recv_sem=recv_sems.at[_FWD_R + slot],
                device_id=(dev,), device_id_type=pl.DeviceIdType.MESH,
            )

        recv_l_top = copy(xb_ref.at[pl.ds(0, half), :], rows(left, half), _R_TOP, right)
        recv_l_top.wait_recv()
        f8s_ref[0] = xg_ref[rows(left, half), :].astype(jnp.float8_e4m3fn)
        fwd_r = f8copy(0, right)
        fwd_r.start()

        recv_r_bot = copy(xb_ref.at[pl.ds(half, half), :], rows(right, half, half), _L_BOT, left)
        recv_r_bot.wait_recv()
        f8s_ref[1] = xg_ref[rows(right, half, half), :].astype(jnp.float8_e4m3fn)
        fwd_l = f8copy(1, left)
        fwd_l.start()

        recv_l_bot = copy(xb_ref.at[pl.ds(half, half), :], rows(left, half, half), _R_BOT, right)
        recv_l_bot.wait_recv()
        gemm(xg_ref[rows(left), :], left)

        recv_r_top = copy(xb_ref.at[pl.ds(0, half), :], rows(right, half), _L_TOP, left)
        recv_r_top.wait_recv()
        gemm(xg_ref[rows(right), :], right)

        recv_opp_top = f8copy(0, right)
        recv_opp_bot = f8copy(1, left)
        recv_opp_top.wait_recv()
        recv_opp_bot.wait_recv()
        xg_ref[rows(opp, half), :] = f8r_ref[0].astype(jnp.bfloat16)
        xg_ref[rows(opp, half, half), :] = f8r_ref[1].astype(jnp.bfloat16)
        gemm(xg_ref[rows(opp), :], opp)

        for s in (s_r_top, s_r_bot, s_l_bot, s_l_top, fwd_r, fwd_l):
            s.wait_send()

    return pl.pallas_call(
        body,
        out_shape=jax.ShapeDtypeStruct((N_DEV * m_per, n_per), jnp.float32),
        in_specs=[
            pl.BlockSpec(memory_space=pl.ANY),
            pl.BlockSpec(memory_space=pl.ANY),
        ],
        out_specs=pl.BlockSpec(memory_space=pltpu.VMEM),
        scratch_shapes=[
            pltpu.VMEM((m_per, k), jnp.float32),
            pltpu.VMEM((k, n_per), jnp.float32),
            pltpu.VMEM((m_per, k), jnp.bfloat16),
            pltpu.VMEM((k, n_per), jnp.bfloat16),
            pltpu.VMEM((N_DEV * m_per, k), jnp.bfloat16),
            pltpu.VMEM((2, m_per // 2, k), jnp.float8_e4m3fn),
            pltpu.VMEM((2, m_per // 2, k), jnp.float8_e4m3fn),
            pltpu.SemaphoreType.DMA((2,)),
            pltpu.SemaphoreType.DMA((6,)),
            pltpu.SemaphoreType.DMA((6,)),
        ],
        compiler_params=pltpu.CompilerParams(collective_id=0),
    )(x, w_mat)


# device time: 15097 ns/iter; 1.0694x vs baseline; 1.0694x over previous
import jax
import jax.numpy as jnp
from jax import lax
from jax.experimental import pallas as pl
from jax.experimental.pallas import tpu as pltpu

N_DEV = 4
_GELU_C = 0.7978845608028654

_R_TOP = 0
_R_BOT = 1
_L_BOT = 2
_L_TOP = 3
_FWD_R = 4
_FWD_L = 5


def _gelu(y):
    return 0.5 * y * (1.0 + jnp.tanh(_GELU_C * (y + 0.044715 * y * y * y)))


def kernel(x, w_mat):
    x = x.astype(jnp.bfloat16)
    m_per, k = x.shape
    _, n_per = w_mat.shape
    half = m_per // 2

    def body(x_ref, w_hbm, out_ref, w32_ref, wb_ref, xg_ref, f8s_ref, f8r_ref,
             cp_sems, send_sems, recv_sems):
        my = lax.axis_index("i")
        left = (my - 1) % N_DEV
        right = (my + 1) % N_DEV
        opp = (my + 2) % N_DEV

        def rows(origin, size=m_per, off=0):
            return pl.ds(origin * m_per + off, size)

        def copy(src, dst_rows, sem, dev):
            return pltpu.make_async_remote_copy(
                src_ref=src, dst_ref=xg_ref.at[dst_rows, :],
                send_sem=send_sems.at[sem], recv_sem=recv_sems.at[sem],
                device_id=(dev,), device_id_type=pl.DeviceIdType.MESH,
            )

        def gemm(x_block, origin):
            y = jnp.dot(x_block, wb_ref[:, :], preferred_element_type=jnp.float32)
            out_ref[rows(origin), :] = _gelu(y)

        cp_w = pltpu.make_async_copy(w_hbm, w32_ref, cp_sems.at[0])
        cp_w.start()

        barrier_sem = pltpu.get_barrier_semaphore()
        for nbr in (left, right):
            pl.semaphore_signal(
                barrier_sem, inc=1,
                device_id=(nbr,), device_id_type=pl.DeviceIdType.MESH,
            )
        pl.semaphore_wait(barrier_sem, 2)

        s_r_top = copy(x_ref.at[pl.ds(0, half), :], rows(my, half), _R_TOP, right)
        s_r_bot = copy(x_ref.at[pl.ds(half, half), :], rows(my, half, half), _R_BOT, right)
        s_l_bot = copy(x_ref.at[pl.ds(half, half), :], rows(my, half, half), _L_BOT, left)
        s_l_top = copy(x_ref.at[pl.ds(0, half), :], rows(my, half), _L_TOP, left)
        s_r_top.start()
        s_l_bot.start()
        s_r_bot.start()
        s_l_top.start()

        cp_w.wait()
        wb_ref[:, :] = w32_ref[:, :].astype(jnp.bfloat16)
        gemm(x_ref[:, :], my)

        def f8copy(slot, dev):
            return pltpu.make_async_remote_copy(
                src_ref=f8s_ref.at[slot], dst_ref=f8r_ref.at[slot],
                send_sem=send_sems.at[_FWD_R + slot],
                recv_sem=recv_sems.at[_FWD_R + slot],
                device_id=(dev,), device_id_type=pl.DeviceIdType.MESH,
            )

        recv_l_top = copy(x_ref.at[pl.ds(0, half), :], rows(left, half), _R_TOP, right)
        recv_l_top.wait_recv()
        f8s_ref[0] = xg_ref[rows(left, half), :].astype(jnp.float8_e4m3fn)
        fwd_r = f8copy(0, right)
        fwd_r.start()

        recv_r_bot = copy(x_ref.at[pl.ds(half, half), :], rows(right, half, half), _L_BOT, left)
        recv_r_bot.wait_recv()
        f8s_ref[1] = xg_ref[rows(right, half, half), :].astype(jnp.float8_e4m3fn)
        fwd_l = f8copy(1, left)
        fwd_l.start()

        recv_l_bot = copy(x_ref.at[pl.ds(half, half), :], rows(left, half, half), _R_BOT, right)
        recv_l_bot.wait_recv()
        gemm(xg_ref[rows(left), :], left)

        recv_r_top = copy(x_ref.at[pl.ds(0, half), :], rows(right, half), _L_TOP, left)
        recv_r_top.wait_recv()
        gemm(xg_ref[rows(right), :], right)

        recv_opp_top = f8copy(0, right)
        recv_opp_bot = f8copy(1, left)
        recv_opp_top.wait_recv()
        recv_opp_bot.wait_recv()
        xg_ref[rows(opp, half), :] = f8r_ref[0].astype(jnp.bfloat16)
        xg_ref[rows(opp, half, half), :] = f8r_ref[1].astype(jnp.bfloat16)
        gemm(xg_ref[rows(opp), :], opp)

        for s in (s_r_top, s_r_bot, s_l_bot, s_l_top, fwd_r, fwd_l):
            s.wait_send()

    return pl.pallas_call(
        body,
        out_shape=jax.ShapeDtypeStruct((N_DEV * m_per, n_per), jnp.float32),
        in_specs=[
            pl.BlockSpec(memory_space=pltpu.VMEM),
            pl.BlockSpec(memory_space=pl.ANY),
        ],
        out_specs=pl.BlockSpec(memory_space=pltpu.VMEM),
        scratch_shapes=[
            pltpu.VMEM((k, n_per), jnp.float32),
            pltpu.VMEM((k, n_per), jnp.bfloat16),
            pltpu.VMEM((N_DEV * m_per, k), x.dtype),
            pltpu.VMEM((2, m_per // 2, k), jnp.float8_e4m3fn),
            pltpu.VMEM((2, m_per // 2, k), jnp.float8_e4m3fn),
            pltpu.SemaphoreType.DMA((1,)),
            pltpu.SemaphoreType.DMA((6,)),
            pltpu.SemaphoreType.DMA((6,)),
        ],
        compiler_params=pltpu.CompilerParams(collective_id=0),
    )(x, w_mat)
